# Initial kernel scaffold; baseline (speedup 1.0000x reference)
import functools

import jax
import jax.numpy as jnp
from jax import lax
from jax.experimental import pallas as pl
from jax.experimental.pallas import tpu as pltpu
from jax.experimental.pallas import tpu_sc as plsc

N = 10000
E = 320000
D = 128
H = 128

NC = 2
NS = 16
NW = NC * NS

NP = 10240
PAD_DST = N + 100
K = 128
EW = E // NW
CH = -(-EW // K)
EWP = CH * K
DEGW = 16

ROWS_PER_TILE = NP // NS


def _sc_degree_body(dst_hbm, zeros_hbm, out_hbm, deg_acc, dst_v, ones_v, csem):
    c = lax.axis_index("c")
    s = lax.axis_index("s")
    wid = s * NC + c

    for i in range(K * DEGW // 16):
        ones_v[pl.ds(i * 16, 16)] = jnp.full((16,), 1.0, jnp.float32)

    pltpu.sync_copy(
        zeros_hbm.at[pl.ds(s * ROWS_PER_TILE, ROWS_PER_TILE)],
        deg_acc.at[pl.ds(s * ROWS_PER_TILE, ROWS_PER_TILE)],
    )
    pltpu.sync_copy(dst_hbm.at[wid], dst_v)
    plsc.subcore_barrier()

    ones2d = ones_v.reshape(K, DEGW)

    def chunk(j, _):
        pltpu.sync_copy(ones2d, deg_acc.at[dst_v.at[j]], add=True)
        return ()

    lax.fori_loop(0, CH, chunk, ())
    plsc.subcore_barrier()

    pltpu.sync_copy(
        deg_acc.at[pl.ds(s * ROWS_PER_TILE, ROWS_PER_TILE)],
        out_hbm.at[c, pl.ds(s * ROWS_PER_TILE, ROWS_PER_TILE)],
    )


@jax.jit
def _sc_degree(dst_tiles, zeros_deg):
    mesh = plsc.VectorSubcoreMesh(core_axis_name="c", subcore_axis_name="s")
    return pl.kernel(
        _sc_degree_body,
        out_type=jax.ShapeDtypeStruct((NC, NP, DEGW), jnp.float32),
        mesh=mesh,
        scratch_types=[
            pltpu.VMEM_SHARED((NP, DEGW), jnp.float32),
            pltpu.VMEM((CH, K), jnp.int32),
            pltpu.VMEM((K * DEGW,), jnp.float32),
            pltpu.SemaphoreType.DMA,
        ],
    )(dst_tiles, zeros_deg)


def _sc_agg_body(hs_hbm, src_hbm, dst_hbm, zeros_hbm, out_hbm,
                 acc, src_v, dst_v, rows_v, gsem):
    c = lax.axis_index("c")
    s = lax.axis_index("s")
    wid = s * NC + c

    pltpu.sync_copy(
        zeros_hbm.at[pl.ds(s * ROWS_PER_TILE, ROWS_PER_TILE)],
        acc.at[pl.ds(s * ROWS_PER_TILE, ROWS_PER_TILE)],
    )
    pltpu.sync_copy(src_hbm.at[wid], src_v)
    pltpu.sync_copy(dst_hbm.at[wid], dst_v)
    plsc.subcore_barrier()

    def chunk(j, _):
        pltpu.async_copy(hs_hbm.at[src_v.at[j]], rows_v, gsem).wait()
        pltpu.sync_copy(rows_v, acc.at[dst_v.at[j]], add=True)
        return ()

    lax.fori_loop(0, CH, chunk, ())
    plsc.subcore_barrier()

    pltpu.sync_copy(
        acc.at[pl.ds(s * ROWS_PER_TILE, ROWS_PER_TILE)],
        out_hbm.at[c, pl.ds(s * ROWS_PER_TILE, ROWS_PER_TILE)],
    )


@jax.jit
def _sc_aggregate(hs, src_tiles, dst_tiles, zeros_rows):
    mesh = plsc.VectorSubcoreMesh(core_axis_name="c", subcore_axis_name="s")
    return pl.kernel(
        _sc_agg_body,
        out_type=jax.ShapeDtypeStruct((NC, NP, H), jnp.float32),
        mesh=mesh,
        scratch_types=[
            pltpu.VMEM_SHARED((NP, H), jnp.float32),
            pltpu.VMEM((CH, K), jnp.int32),
            pltpu.VMEM((CH, K), jnp.int32),
            pltpu.VMEM((K, H), jnp.float32),
            pltpu.SemaphoreType.DMA,
        ],
    )(hs, src_tiles, dst_tiles, zeros_rows)


BLK = 512
GRID = NP // BLK


def _tc1_body(x_ref, wg_ref, bg_ref, w1_ref, dega_ref, degb_ref, out_ref):
    xb = x_ref[...]
    g = jax.nn.sigmoid(
        jnp.dot(xb, wg_ref[...], preferred_element_type=jnp.float32)
        + bg_ref[...]
    )
    h = jnp.dot(xb * g, w1_ref[...], preferred_element_type=jnp.float32)
    deg = dega_ref[...] + degb_ref[...] + 1.0
    out_ref[...] = h * lax.rsqrt(deg)


def _tc2_body(acc_ref, hs_ref, b1_ref, w2_ref, dega_ref, degb_ref, out_ref):
    deg = dega_ref[...] + degb_ref[...] + 1.0
    dinv = lax.rsqrt(deg)
    pre = (acc_ref[0] + acc_ref[1] + hs_ref[...]) * dinv + b1_ref[...]
    o1 = jnp.maximum(pre, 0.0)
    h2 = jnp.dot(o1, w2_ref[...], preferred_element_type=jnp.float32)
    out_ref[...] = h2 * dinv


def _tc3_body(acc_ref, hs_ref, b2_ref, dega_ref, degb_ref, out_ref):
    deg = dega_ref[...] + degb_ref[...] + 1.0
    dinv = lax.rsqrt(deg)
    out_ref[...] = (acc_ref[0] + acc_ref[1] + hs_ref[...]) * dinv + b2_ref[...]


BLK = 512
GRID = NP // BLK

_row_spec = pl.BlockSpec((BLK, D), lambda i: (i, 0))
_deg_spec = pl.BlockSpec((BLK, 1), lambda i: (i, 0))
_full_spec = pl.BlockSpec((D, H), lambda i: (0, 0))
_bias_spec = pl.BlockSpec((1, H), lambda i: (0, 0))
_acc_spec = pl.BlockSpec((NC, BLK, H), lambda i: (0, i, 0))


@jax.jit
def _tc_stage1(xp, Wg, bg, W1, dega, degb):
    return pl.pallas_call(
        _tc1_body,
        grid=(GRID,),
        in_specs=[_row_spec, _full_spec, _bias_spec, _full_spec,
                  _deg_spec, _deg_spec],
        out_specs=_row_spec,
        out_shape=jax.ShapeDtypeStruct((NP, H), jnp.float32),
    )(xp, Wg, bg.reshape(1, D), W1, dega, degb)


@jax.jit
def _tc_stage2(acc, hs1, b1, W2, dega, degb):
    return pl.pallas_call(
        _tc2_body,
        grid=(GRID,),
        in_specs=[_acc_spec, _row_spec, _bias_spec, _full_spec,
                  _deg_spec, _deg_spec],
        out_specs=_row_spec,
        out_shape=jax.ShapeDtypeStruct((NP, H), jnp.float32),
    )(acc, hs1, b1.reshape(1, H), W2, dega, degb)


@jax.jit
def _tc_stage3(acc, hs2, b2, dega, degb):
    return pl.pallas_call(
        _tc3_body,
        grid=(GRID,),
        in_specs=[_acc_spec, _row_spec, _bias_spec, _deg_spec, _deg_spec],
        out_specs=_row_spec,
        out_shape=jax.ShapeDtypeStruct((NP, H), jnp.float32),
    )(acc, hs2, b2.reshape(1, H), dega, degb)


def kernel(x, edge_index, Wg, bg, W1, b1, W2, b2):
    src = edge_index[0].astype(jnp.int32)
    dst = edge_index[1].astype(jnp.int32)

    pad = NW * EWP - E
    srcp = jnp.concatenate([src, jnp.zeros((pad,), jnp.int32)])
    dstp = jnp.concatenate([dst, jnp.full((pad,), PAD_DST, jnp.int32)])
    src_tiles = srcp.reshape(NW, CH, K)
    dst_tiles = dstp.reshape(NW, CH, K)

    xp = jnp.zeros((NP, D), jnp.float32).at[:N].set(x)
    zeros_deg = jnp.zeros((NP, DEGW), jnp.float32)
    zeros_rows = jnp.zeros((NP, H), jnp.float32)

    degp = _sc_degree(dst_tiles, zeros_deg)
    dega = degp[0, :, :1]
    degb = degp[1, :, :1]

    hs1 = _tc_stage1(xp, Wg, bg, W1, dega, degb)
    acc1 = _sc_aggregate(hs1, src_tiles, dst_tiles, zeros_rows)
    hs2 = _tc_stage2(acc1, hs1, b1, W2, dega, degb)
    acc2 = _sc_aggregate(hs2, src_tiles, dst_tiles, zeros_rows)
    out = _tc_stage3(acc2, hs2, b2, dega, degb)
    return out[:N]


# speedup vs baseline: 10.6268x; 10.6268x over previous
import functools

import jax
import jax.numpy as jnp
from jax import lax
from jax.experimental import pallas as pl
from jax.experimental.pallas import tpu as pltpu
from jax.experimental.pallas import tpu_sc as plsc

N = 10000
E = 320000
D = 128
H = 128

NC = 2
NS = 16
NW = NC * NS

NP = 10240
PAD_DST = N + 100
K = 128
EW = E // NW
CH = -(-EW // K)
EWP = CH * K
DEGW = 128

ROWS_PER_TILE = NP // NS


def _sc_degree_body(dst_hbm, zeros_hbm, out_hbm, deg_acc, dst_v, ones_v):
    c = lax.axis_index("c")
    s = lax.axis_index("s")
    wid = s * NC + c

    def fill(r, _):
        for i in range(DEGW // 16):
            ones_v[r, pl.ds(i * 16, 16)] = jnp.full((16,), 1.0, jnp.float32)
        return ()

    lax.fori_loop(0, K, fill, ())

    pltpu.sync_copy(
        zeros_hbm.at[pl.ds(s * ROWS_PER_TILE, ROWS_PER_TILE)],
        deg_acc.at[pl.ds(s * ROWS_PER_TILE, ROWS_PER_TILE)],
    )
    pltpu.sync_copy(dst_hbm.at[wid], dst_v)
    plsc.subcore_barrier()

    def chunk(j, _):
        pltpu.sync_copy(ones_v, deg_acc.at[dst_v.at[j]], add=True)
        return ()

    lax.fori_loop(0, CH, chunk, ())
    plsc.subcore_barrier()

    pltpu.sync_copy(
        deg_acc.at[pl.ds(s * ROWS_PER_TILE, ROWS_PER_TILE)],
        out_hbm.at[c, pl.ds(s * ROWS_PER_TILE, ROWS_PER_TILE)],
    )


@jax.jit
def _sc_degree(dst_tiles, zeros_deg):
    mesh = plsc.VectorSubcoreMesh(core_axis_name="c", subcore_axis_name="s")
    return pl.kernel(
        _sc_degree_body,
        out_type=jax.ShapeDtypeStruct((NC, NP, DEGW), jnp.float32),
        mesh=mesh,
        scratch_types=[
            pltpu.VMEM_SHARED((NP, DEGW), jnp.float32),
            pltpu.VMEM((CH, K), jnp.int32),
            pltpu.VMEM((K, DEGW), jnp.float32),
        ],
    )(dst_tiles, zeros_deg)


def _sc_agg_body(hs_hbm, src_hbm, dst_hbm, zeros_hbm, out_hbm,
                 acc, src_v, dst_v, rows_v, gsem):
    c = lax.axis_index("c")
    s = lax.axis_index("s")
    wid = s * NC + c

    pltpu.sync_copy(
        zeros_hbm.at[pl.ds(s * ROWS_PER_TILE, ROWS_PER_TILE)],
        acc.at[pl.ds(s * ROWS_PER_TILE, ROWS_PER_TILE)],
    )
    pltpu.sync_copy(src_hbm.at[wid], src_v)
    pltpu.sync_copy(dst_hbm.at[wid], dst_v)
    plsc.subcore_barrier()

    def chunk(j, _):
        pltpu.async_copy(hs_hbm.at[src_v.at[j]], rows_v, gsem).wait()
        pltpu.sync_copy(rows_v, acc.at[dst_v.at[j]], add=True)
        return ()

    lax.fori_loop(0, CH, chunk, ())
    plsc.subcore_barrier()

    pltpu.sync_copy(
        acc.at[pl.ds(s * ROWS_PER_TILE, ROWS_PER_TILE)],
        out_hbm.at[c, pl.ds(s * ROWS_PER_TILE, ROWS_PER_TILE)],
    )


@jax.jit
def _sc_aggregate(hs, src_tiles, dst_tiles, zeros_rows):
    mesh = plsc.VectorSubcoreMesh(core_axis_name="c", subcore_axis_name="s")
    return pl.kernel(
        _sc_agg_body,
        out_type=jax.ShapeDtypeStruct((NC, NP, H), jnp.float32),
        mesh=mesh,
        scratch_types=[
            pltpu.VMEM_SHARED((NP, H), jnp.float32),
            pltpu.VMEM((CH, K), jnp.int32),
            pltpu.VMEM((CH, K), jnp.int32),
            pltpu.VMEM((K, H), jnp.float32),
            pltpu.SemaphoreType.DMA,
        ],
    )(hs, src_tiles, dst_tiles, zeros_rows)


BLK = 512
GRID = NP // BLK


def _tc1_body(x_ref, wg_ref, bg_ref, w1_ref, dega_ref, degb_ref, out_ref):
    xb = x_ref[...]
    g = jax.nn.sigmoid(
        jnp.dot(xb, wg_ref[...], preferred_element_type=jnp.float32)
        + bg_ref[...]
    )
    h = jnp.dot(xb * g, w1_ref[...], preferred_element_type=jnp.float32)
    deg = dega_ref[...] + degb_ref[...] + 1.0
    out_ref[...] = h * lax.rsqrt(deg)


def _tc2_body(acc_ref, hs_ref, b1_ref, w2_ref, dega_ref, degb_ref, out_ref):
    deg = dega_ref[...] + degb_ref[...] + 1.0
    dinv = lax.rsqrt(deg)
    pre = (acc_ref[0] + acc_ref[1] + hs_ref[...]) * dinv + b1_ref[...]
    o1 = jnp.maximum(pre, 0.0)
    h2 = jnp.dot(o1, w2_ref[...], preferred_element_type=jnp.float32)
    out_ref[...] = h2 * dinv


def _tc3_body(acc_ref, hs_ref, b2_ref, dega_ref, degb_ref, out_ref):
    deg = dega_ref[...] + degb_ref[...] + 1.0
    dinv = lax.rsqrt(deg)
    out_ref[...] = (acc_ref[0] + acc_ref[1] + hs_ref[...]) * dinv + b2_ref[...]


BLK = 512
GRID = NP // BLK

_row_spec = pl.BlockSpec((BLK, D), lambda i: (i, 0))
_deg_spec = pl.BlockSpec((BLK, 1), lambda i: (i, 0))
_full_spec = pl.BlockSpec((D, H), lambda i: (0, 0))
_bias_spec = pl.BlockSpec((1, H), lambda i: (0, 0))
_acc_spec = pl.BlockSpec((NC, BLK, H), lambda i: (0, i, 0))


@jax.jit
def _tc_stage1(xp, Wg, bg, W1, dega, degb):
    return pl.pallas_call(
        _tc1_body,
        grid=(GRID,),
        in_specs=[_row_spec, _full_spec, _bias_spec, _full_spec,
                  _deg_spec, _deg_spec],
        out_specs=_row_spec,
        out_shape=jax.ShapeDtypeStruct((NP, H), jnp.float32),
    )(xp, Wg, bg.reshape(1, D), W1, dega, degb)


@jax.jit
def _tc_stage2(acc, hs1, b1, W2, dega, degb):
    return pl.pallas_call(
        _tc2_body,
        grid=(GRID,),
        in_specs=[_acc_spec, _row_spec, _bias_spec, _full_spec,
                  _deg_spec, _deg_spec],
        out_specs=_row_spec,
        out_shape=jax.ShapeDtypeStruct((NP, H), jnp.float32),
    )(acc, hs1, b1.reshape(1, H), W2, dega, degb)


@jax.jit
def _tc_stage3(acc, hs2, b2, dega, degb):
    return pl.pallas_call(
        _tc3_body,
        grid=(GRID,),
        in_specs=[_acc_spec, _row_spec, _bias_spec, _deg_spec, _deg_spec],
        out_specs=_row_spec,
        out_shape=jax.ShapeDtypeStruct((NP, H), jnp.float32),
    )(acc, hs2, b2.reshape(1, H), dega, degb)


def kernel(x, edge_index, Wg, bg, W1, b1, W2, b2):
    src = edge_index[0].astype(jnp.int32)
    dst = edge_index[1].astype(jnp.int32)

    pad = NW * EWP - E
    srcp = jnp.concatenate([src, jnp.zeros((pad,), jnp.int32)])
    dstp = jnp.concatenate([dst, jnp.full((pad,), PAD_DST, jnp.int32)])
    src_tiles = srcp.reshape(NW, CH, K)
    dst_tiles = dstp.reshape(NW, CH, K)

    xp = jnp.zeros((NP, D), jnp.float32).at[:N].set(x)
    zeros_rows = jnp.zeros((NP, H), jnp.float32)

    degp = _sc_degree(dst_tiles, zeros_rows)
    dega = degp[0, :, :1]
    degb = degp[1, :, :1]

    hs1 = _tc_stage1(xp, Wg, bg, W1, dega, degb)
    acc1 = _sc_aggregate(hs1, src_tiles, dst_tiles, zeros_rows)
    hs2 = _tc_stage2(acc1, hs1, b1, W2, dega, degb)
    acc2 = _sc_aggregate(hs2, src_tiles, dst_tiles, zeros_rows)
    out = _tc_stage3(acc2, hs2, b2, dega, degb)
    return out[:N]
